# Initial kernel scaffold; baseline (speedup 1.0000x reference)
import jax
import jax.numpy as jnp
from jax.experimental import pallas as pl


def kernel(bias_0, bias_1):
    raise NotImplementedError("write your pallas kernel here")


# speedup vs baseline: 6.4254x; 6.4254x over previous
import functools

import jax
import jax.numpy as jnp
from jax.experimental import pallas as pl
from jax.experimental.pallas import tpu as pltpu

_L = 32
_H = 16
_T = _L * _L


def _expand_body(b0_ref, b1_ref, e0_ref, e1_ref):
    w = pl.program_id(0)
    j = jax.lax.broadcasted_iota(jnp.int32, (2 * _L, _T), 0)
    k = jax.lax.broadcasted_iota(jnp.int32, (2 * _L, _T), 1)
    m0 = (j == (k // _L) + _L - w).astype(jnp.float32)
    m1 = (j == (k % _L) + _L - w).astype(jnp.float32)
    e0_ref[0] = jnp.dot(b0_ref[...], m0, preferred_element_type=jnp.float32)
    e1_ref[0] = jnp.dot(b1_ref[...], m1, preferred_element_type=jnp.float32)


def _add_body(e0_ref, e1_ref, out_ref):
    e0 = e0_ref[...]
    e1 = e1_ref[...]
    out_ref[...] = e0[:, None, :, :] + e1[None, ...]


@jax.jit
def kernel(bias_0, bias_1):
    expand = pl.pallas_call(
        _expand_body,
        grid=(_L,),
        in_specs=[
            pl.BlockSpec((_H, 2 * _L), lambda w: (0, 0)),
            pl.BlockSpec((_H, 2 * _L), lambda w: (0, 0)),
        ],
        out_specs=[
            pl.BlockSpec((1, _H, _T), lambda w: (w, 0, 0)),
            pl.BlockSpec((1, _H, _T), lambda w: (w, 0, 0)),
        ],
        out_shape=[
            jax.ShapeDtypeStruct((_L, _H, _T), jnp.float32),
            jax.ShapeDtypeStruct((_L, _H, _T), jnp.float32),
        ],
    )
    e0, e1 = expand(bias_0, bias_1)

    add = pl.pallas_call(
        _add_body,
        grid=(_L,),
        in_specs=[
            pl.BlockSpec((1, _H, _T), lambda i: (i, 0, 0)),
            pl.BlockSpec((_L, _H, _T), lambda i: (0, 0, 0)),
        ],
        out_specs=pl.BlockSpec((1, _L, _H, _T), lambda i: (i, 0, 0, 0)),
        out_shape=jax.ShapeDtypeStruct((_L, _L, _H, _T), jnp.float32),
    )
    out = add(e0, e1)
    return out.reshape(_T, _H, _T)
